# Initial kernel scaffold; baseline (speedup 1.0000x reference)
import jax
import jax.numpy as jnp
from jax.experimental import pallas as pl


def kernel(x, edge_index, W0, b0, W1, b1, W2, b2, A0w, A0b, A1w, A1b, A2w, A2b):
    raise NotImplementedError("write your pallas kernel here")


# speedup vs baseline: 78.5909x; 78.5909x over previous
import functools

import jax
import jax.numpy as jnp
from jax import lax
from jax.experimental import pallas as pl
from jax.experimental.pallas import tpu as pltpu
from jax.experimental.pallas import tpu_sc as plsc

N = 10000
NH = 4
F = 64
L = 16
NC, NS = 2, 16
NW = NC * NS
NPW = 320
NPAD = NW * NPW
BB = 128
TROW = 128
BN = 1000


def _tc_dense(xin, W, b2d, wq):
    Nn, Fin = xin.shape
    grid = Nn // BN

    def body(x_ref, w_ref, b_ref, wq_ref, h_ref, q_ref, m_ref, macc):
        i = pl.program_id(0)
        h = jnp.dot(x_ref[...], w_ref[...].T, preferred_element_type=jnp.float32)
        h = h + b_ref[...]
        h = jnp.where(h >= 0.0, h, 0.2 * h)
        h_ref[...] = h
        q = jnp.dot(h, wq_ref[...].T, preferred_element_type=jnp.float32)
        q_ref[...] = q
        bm = jnp.max(q, axis=0, keepdims=True)

        @pl.when(i == 0)
        def _():
            macc[0:1, 0:8] = bm

        @pl.when(i > 0)
        def _():
            macc[0:1, 0:8] = jnp.maximum(macc[0:1, 0:8], bm)

        @pl.when(i == grid - 1)
        def _():
            m_ref[...] = macc[0:1, 0:8]

    return pl.pallas_call(
        body,
        grid=(grid,),
        in_specs=[
            pl.BlockSpec((BN, Fin), lambda i: (i, 0)),
            pl.BlockSpec((F, Fin), lambda i: (0, 0)),
            pl.BlockSpec((1, F), lambda i: (0, 0)),
            pl.BlockSpec((8, F), lambda i: (0, 0)),
        ],
        out_specs=[
            pl.BlockSpec((BN, F), lambda i: (i, 0)),
            pl.BlockSpec((BN, 8), lambda i: (i, 0)),
            pl.BlockSpec((1, 8), lambda i: (0, 0)),
        ],
        out_shape=[
            jax.ShapeDtypeStruct((Nn, F), jnp.float32),
            jax.ShapeDtypeStruct((Nn, 8), jnp.float32),
            jax.ShapeDtypeStruct((1, 8), jnp.float32),
        ],
        scratch_shapes=[pltpu.VMEM((8, 128), jnp.float32)],
    )(xin, W, b2d, wq)


def _tc_table(h, q, M):
    Nn = h.shape[0]
    grid = Nn // BN

    def body(h_ref, q_ref, m_ref, t_ref):
        g = jnp.exp(q_ref[...] - m_ref[...])
        p0 = jnp.broadcast_to(g[:, 0:1], (BN, L))
        p1 = jnp.broadcast_to(g[:, 1:2], (BN, L))
        pad = jnp.zeros((BN, TROW - 2 * L - F), jnp.float32)
        t_ref[...] = jnp.concatenate([p0, p1, h_ref[...], pad], axis=1)

    return pl.pallas_call(
        body,
        grid=(grid,),
        in_specs=[
            pl.BlockSpec((BN, F), lambda i: (i, 0)),
            pl.BlockSpec((BN, 8), lambda i: (i, 0)),
            pl.BlockSpec((1, 8), lambda i: (0, 0)),
        ],
        out_specs=pl.BlockSpec((BN, TROW), lambda i: (i, 0)),
        out_shape=jax.ShapeDtypeStruct((Nn, TROW), jnp.float32),
    )(h, q, M)


def _sc_edge(table, dstp, srcp, est, final):
    OC = F if final else 3 * F
    NQ = OC // L
    mesh = plsc.VectorSubcoreMesh(core_axis_name="c", subcore_axis_name="s")

    @functools.partial(
        pl.kernel,
        out_type=jax.ShapeDtypeStruct((NPAD * OC,), jnp.float32),
        mesh=mesh,
        scratch_types=[
            pltpu.VMEM((BB,), jnp.int32),
            pltpu.VMEM((BB, TROW), jnp.float32),
            pltpu.VMEM((BB + L,), jnp.int32),
            pltpu.VMEM((48,), jnp.int32),
            pltpu.VMEM((NPW * OC,), jnp.float32),
            pltpu.SemaphoreType.DMA,
        ],
    )
    def k(table_hbm, dst_hbm, src_hbm, est_hbm, out_hbm,
          idx_v, stage_v, src_v, est_v, outb, sem):
        wid = lax.axis_index("c") * NS + lax.axis_index("s")
        n0 = wid * NPW
        pltpu.sync_copy(est_hbm, est_v)
        e0 = est_v[pl.ds(wid, L)][0]
        e1 = est_v[pl.ds(wid + 1, L)][0]
        e0a = (e0 // 8) * 8
        joff = e0 - e0a
        cnt = e1 - e0a
        nb = lax.div(cnt + BB - 1, BB)
        zero = jnp.zeros((L,), jnp.float32)
        onev = jnp.ones((L,), jnp.float32)

        def zero_rows(lo, hi):
            def zb(r, carry):
                for qq in range(NQ):
                    outb[pl.ds(r * OC + qq * L, L)] = zero
                return carry
            lax.fori_loop(lo, jnp.maximum(hi, lo), zb, 0)

        def flush(cur, nr, accu, acc2, acc3, s2, s3, deg):
            row = cur - n0
            zero_rows(nr, row)

            @pl.when(cur >= 0)
            def _():
                base = row * OC
                if final:
                    for qq in range(4):
                        v = (2.0 * accu[qq] / deg + acc2[qq] / s2
                             + acc3[qq] / s3) * 0.25
                        outb[pl.ds(base + qq * L, L)] = jnp.maximum(v, 0.0)
                else:
                    for qq in range(4):
                        outb[pl.ds(base + qq * L, L)] = jnp.maximum(
                            accu[qq] / deg, 0.0)
                        outb[pl.ds(base + F + qq * L, L)] = jnp.maximum(
                            acc2[qq] / s2, 0.0)
                        outb[pl.ds(base + 2 * F + qq * L, L)] = jnp.maximum(
                            acc3[qq] / s3, 0.0)
            return jnp.maximum(row + 1, nr)

        zacc = (tuple(zero for _ in range(4)), tuple(zero for _ in range(4)),
                tuple(zero for _ in range(4)), zero, zero, zero)

        def batch_body(b, carry):
            cur, nr, accu, acc2, acc3, s2, s3, deg = carry
            eb = e0a + b * BB
            pltpu.sync_copy(dst_hbm.at[pl.ds(eb, BB)], idx_v)
            pltpu.sync_copy(src_hbm.at[pl.ds(eb, BB)], src_v.at[pl.ds(0, BB)])
            pltpu.async_copy(table_hbm.at[idx_v], stage_v, sem).wait()
            js = jnp.where(b == 0, joff, 0)
            je = jnp.minimum(cnt - b * BB, BB)

            def edge_body(j, ec):
                cur, nr, accu, acc2, acc3, s2, s3, deg = ec
                sid = src_v[pl.ds(j, L)][0]

                def do_flush(op):
                    cur0, nr0, au, a2, a3, ss2, ss3, dd = op
                    nr1 = flush(cur0, nr0, au, a2, a3, ss2, ss3, dd)
                    return (nr1,) + zacc

                def no_flush(op):
                    cur0, nr0, au, a2, a3, ss2, ss3, dd = op
                    return (nr0, au, a2, a3, ss2, ss3, dd)

                nr, accu, acc2, acc3, s2, s3, deg = lax.cond(
                    sid != cur, do_flush, no_flush,
                    (cur, nr, accu, acc2, acc3, s2, s3, deg))
                rg2 = stage_v[j, pl.ds(0, L)]
                rg3 = stage_v[j, pl.ds(L, L)]
                rh = [stage_v[j, pl.ds(2 * L + qq * L, L)] for qq in range(4)]
                accu = tuple(accu[qq] + rh[qq] for qq in range(4))
                acc2 = tuple(acc2[qq] + rg2 * rh[qq] for qq in range(4))
                acc3 = tuple(acc3[qq] + rg3 * rh[qq] for qq in range(4))
                s2 = s2 + rg2
                s3 = s3 + rg3
                deg = deg + onev
                return (sid, nr, accu, acc2, acc3, s2, s3, deg)

            return lax.fori_loop(js, je, edge_body,
                                 (cur, nr, accu, acc2, acc3, s2, s3, deg))

        init = (jnp.int32(-1), jnp.int32(0)) + zacc
        cur, nr, accu, acc2, acc3, s2, s3, deg = lax.fori_loop(
            0, nb, batch_body, init)
        nr = flush(cur, nr, accu, acc2, acc3, s2, s3, deg)
        zero_rows(nr, NPW)
        pltpu.sync_copy(outb, out_hbm.at[pl.ds(n0 * OC, NPW * OC)])

    return k(table, dstp, srcp, est)


def _merge_heads(W):
    Wr = W.reshape(-1, F, NH)
    return jnp.concatenate([Wr[:, :, 0] + Wr[:, :, 1], Wr[:, :, 2], Wr[:, :, 3]],
                           axis=1)


def _wq(Aw):
    w = Aw[2:4, :F] + Aw[2:4, F:]
    return jnp.pad(w, ((0, 6), (0, 0)))


def kernel(x, edge_index, W0, b0, W1, b1, W2, b2,
           A0w, A0b, A1w, A1b, A2w, A2b):
    src = edge_index[0].astype(jnp.int32)
    dst = edge_index[1].astype(jnp.int32)
    E = src.shape[0]
    bounds = jnp.arange(NW + 1, dtype=jnp.int32) * NPW
    est = jnp.searchsorted(src, bounds, side="left").astype(jnp.int32)
    est = jnp.pad(est, (0, 48 - (NW + 1)), constant_values=E)
    dstp = jnp.pad(dst, (0, BB))
    srcp = jnp.pad(src, (0, BB), constant_values=-1)

    layers = (
        (x, W0, b0, A0w, False),
        (None, _merge_heads(W1), b1, A1w, False),
        (None, _merge_heads(W2), b2, A2w, True),
    )
    xin = x
    out = None
    for x0, W, b, Aw, final in layers:
        h, q, M = _tc_dense(xin, W, b.reshape(1, F), _wq(Aw))
        T = _tc_table(h, q, M)
        oc = F if final else 3 * F
        out = _sc_edge(T, dstp, srcp, est, final).reshape(NPAD, oc)
        xin = out[:N]
    return xin


# speedup vs baseline: 87.8535x; 1.1179x over previous
import functools

import jax
import jax.numpy as jnp
from jax import lax
from jax.experimental import pallas as pl
from jax.experimental.pallas import tpu as pltpu
from jax.experimental.pallas import tpu_sc as plsc

N = 10000
NH = 4
F = 64
L = 16
NC, NS = 2, 16
NW = NC * NS
NPW = 320
NPAD = NW * NPW
BB = 128
TROW = 128
ACC = 224
BN = 1000


def _tc_dense(xin, W, b2d, wq):
    Fin = xin.shape[1]
    grid = N // BN

    def body(x_ref, w_ref, b_ref, wq_ref, h_ref, q_ref, m_ref, macc):
        i = pl.program_id(0)
        h = jnp.dot(x_ref[...], w_ref[...].T, preferred_element_type=jnp.float32)
        h = h + b_ref[...]
        h = jnp.where(h >= 0.0, h, 0.2 * h)
        h_ref[...] = h
        q = jnp.dot(h, wq_ref[...].T, preferred_element_type=jnp.float32)
        q_ref[...] = q
        bm = jnp.max(q, axis=0, keepdims=True)

        @pl.when(i == 0)
        def _():
            macc[0:1, 0:8] = bm

        @pl.when(i > 0)
        def _():
            macc[0:1, 0:8] = jnp.maximum(macc[0:1, 0:8], bm)

        @pl.when(i == grid - 1)
        def _():
            m_ref[...] = macc[0:1, 0:8]

    return pl.pallas_call(
        body,
        grid=(grid,),
        in_specs=[
            pl.BlockSpec((BN, Fin), lambda i: (i, 0)),
            pl.BlockSpec((F, Fin), lambda i: (0, 0)),
            pl.BlockSpec((1, F), lambda i: (0, 0)),
            pl.BlockSpec((8, F), lambda i: (0, 0)),
        ],
        out_specs=[
            pl.BlockSpec((BN, F), lambda i: (i, 0)),
            pl.BlockSpec((BN, 8), lambda i: (i, 0)),
            pl.BlockSpec((1, 8), lambda i: (0, 0)),
        ],
        out_shape=[
            jax.ShapeDtypeStruct((N, F), jnp.float32),
            jax.ShapeDtypeStruct((N, 8), jnp.float32),
            jax.ShapeDtypeStruct((1, 8), jnp.float32),
        ],
        scratch_shapes=[pltpu.VMEM((8, 128), jnp.float32)],
    )(xin, W, b2d, wq)


def _tc_table(h, q, M):
    grid = N // BN

    def body(h_ref, q_ref, m_ref, t_ref):
        g = jnp.exp(q_ref[...] - m_ref[...])
        p0 = jnp.broadcast_to(g[:, 0:1], (BN, L))
        p1 = jnp.broadcast_to(g[:, 1:2], (BN, L))
        pad = jnp.zeros((BN, TROW - 2 * L - F), jnp.float32)
        t_ref[...] = jnp.concatenate([p0, p1, h_ref[...], pad], axis=1)

    return pl.pallas_call(
        body,
        grid=(grid,),
        in_specs=[
            pl.BlockSpec((BN, F), lambda i: (i, 0)),
            pl.BlockSpec((BN, 8), lambda i: (i, 0)),
            pl.BlockSpec((1, 8), lambda i: (0, 0)),
        ],
        out_specs=pl.BlockSpec((BN, TROW), lambda i: (i, 0)),
        out_shape=jax.ShapeDtypeStruct((N, TROW), jnp.float32),
    )(h, q, M)


def _sc_edge(table, dstp, soffp, est, final):
    OC = F if final else ACC
    mesh = plsc.VectorSubcoreMesh(core_axis_name="c", subcore_axis_name="s")

    @functools.partial(
        pl.kernel,
        out_type=jax.ShapeDtypeStruct((NPAD * OC,), jnp.float32),
        mesh=mesh,
        scratch_types=[
            pltpu.VMEM((2, BB), jnp.int32),
            pltpu.VMEM((2, BB, TROW), jnp.float32),
            pltpu.VMEM((2 * (BB + L),), jnp.int32),
            pltpu.VMEM((48,), jnp.int32),
            pltpu.VMEM((NPW * ACC,), jnp.float32),
            pltpu.VMEM((NPW * F if final else L,), jnp.float32),
            pltpu.SemaphoreType.DMA((2,)),
        ],
    )
    def k(table_hbm, dst_hbm, soff_hbm, est_hbm, out_hbm,
          idx_v, stage_v, soff_v, est_v, outb, outc, sem):
        wid = lax.axis_index("c") * NS + lax.axis_index("s")
        n0 = wid * NPW
        pltpu.sync_copy(est_hbm, est_v)
        e0 = est_v[pl.ds(wid, L)][0]
        e1 = est_v[pl.ds(wid + 1, L)][0]
        e0a = (e0 // 8) * 8
        joff = e0 - e0a
        cnt = e1 - e0a
        nb = lax.div(cnt + BB - 1, BB)
        zero = jnp.zeros((L,), jnp.float32)
        io = lax.iota(jnp.int32, L)

        def zrow(r, c):
            for qq in range(ACC // L):
                outb[pl.ds(r * ACC + qq * L, L)] = zero
            return c
        lax.fori_loop(0, NPW, zrow, 0)

        def gdesc(b):
            slot = lax.rem(b, 2)
            return pltpu.make_async_copy(
                table_hbm.at[idx_v.at[slot]], stage_v.at[slot], sem.at[slot])

        def fire(b):
            slot = lax.rem(b, 2)
            eb = e0a + b * BB
            pltpu.sync_copy(dst_hbm.at[pl.ds(eb, BB)], idx_v.at[slot])
            pltpu.sync_copy(soff_hbm.at[pl.ds(eb, BB)],
                            soff_v.at[pl.ds(slot * (BB + L), BB)])
            gdesc(b).start()

        @pl.when(nb > 0)
        def _():
            fire(0)

        def batch(b, c):
            slot = lax.rem(b, 2)
            gdesc(b).wait()

            @pl.when(b + 1 < nb)
            def _():
                fire(b + 1)

            js = jnp.where(b == 0, joff, 0)
            je = jnp.minimum(cnt - b * BB, BB)

            def edge(j, c2):
                base = soff_v[pl.ds(slot * (BB + L) + j, L)][0]
                rg2 = stage_v[slot, j, pl.ds(0, L)]
                rg3 = stage_v[slot, j, pl.ds(L, L)]
                sv = jnp.where(io == 0, rg2,
                               jnp.where(io == 1, rg3,
                                         jnp.where(io == 2, 1.0, 0.0)))
                for qq in range(4):
                    rh = stage_v[slot, j, pl.ds(2 * L + qq * L, L)]
                    plsc.addupdate(outb.at[pl.ds(base + qq * L, L)], rh)
                    plsc.addupdate(
                        outb.at[pl.ds(base + F + qq * L, L)], rg2 * rh)
                    plsc.addupdate(
                        outb.at[pl.ds(base + 2 * F + qq * L, L)], rg3 * rh)
                plsc.addupdate(outb.at[pl.ds(base + 3 * F, L)], sv)
                return c2

            lax.fori_loop(js, je, edge, c)
            return c

        lax.fori_loop(0, nb, batch, 0)

        lane0 = jnp.zeros((L,), jnp.int32)
        lane1 = jnp.full((L,), 1, jnp.int32)
        lane2 = jnp.full((L,), 2, jnp.int32)

        def nrow(r, c):
            base = r * ACC
            sv = outb[pl.ds(base + 3 * F, L)]
            s2 = jnp.maximum(sv.at[lane0].get(mode="promise_in_bounds"),
                             1e-30)
            s3 = jnp.maximum(sv.at[lane1].get(mode="promise_in_bounds"),
                             1e-30)
            dg = jnp.maximum(sv.at[lane2].get(mode="promise_in_bounds"),
                             1.0)
            if final:
                for qq in range(4):
                    au = outb[pl.ds(base + qq * L, L)]
                    a2 = outb[pl.ds(base + F + qq * L, L)]
                    a3 = outb[pl.ds(base + 2 * F + qq * L, L)]
                    v = (2.0 * au / dg + a2 / s2 + a3 / s3) * 0.25
                    outc[pl.ds(r * F + qq * L, L)] = jnp.maximum(v, 0.0)
            else:
                for qq in range(4):
                    o0 = base + qq * L
                    outb[pl.ds(o0, L)] = jnp.maximum(
                        outb[pl.ds(o0, L)] / dg, 0.0)
                    outb[pl.ds(o0 + F, L)] = jnp.maximum(
                        outb[pl.ds(o0 + F, L)] / s2, 0.0)
                    outb[pl.ds(o0 + 2 * F, L)] = jnp.maximum(
                        outb[pl.ds(o0 + 2 * F, L)] / s3, 0.0)
            return c
        lax.fori_loop(0, NPW, nrow, 0)

        if final:
            pltpu.sync_copy(outc, out_hbm.at[pl.ds(n0 * F, NPW * F)])
        else:
            pltpu.sync_copy(outb, out_hbm.at[pl.ds(n0 * ACC, NPW * ACC)])

    return k(table, dstp, soffp, est)


def _merge_heads(W):
    Wr = W.reshape(-1, F, NH)
    Wm = jnp.concatenate(
        [Wr[:, :, 0] + Wr[:, :, 1], Wr[:, :, 2], Wr[:, :, 3]], axis=1)
    return jnp.pad(Wm, ((0, 0), (0, ACC - 3 * F)))


def _wq(Aw):
    w = Aw[2:4, :F] + Aw[2:4, F:]
    return jnp.pad(w, ((0, 6), (0, 0)))


def kernel(x, edge_index, W0, b0, W1, b1, W2, b2,
           A0w, A0b, A1w, A1b, A2w, A2b):
    src = edge_index[0].astype(jnp.int32)
    dst = edge_index[1].astype(jnp.int32)
    E = src.shape[0]
    bounds = jnp.arange(NW + 1, dtype=jnp.int32) * NPW
    est = jnp.searchsorted(src, bounds, side="left").astype(jnp.int32)
    est = jnp.pad(est, (0, 48 - (NW + 1)), constant_values=E)
    dstp = jnp.pad(dst, (0, BB))
    soffp = jnp.pad((src % NPW) * ACC, (0, BB))

    layers = (
        (W0, b0, A0w, False),
        (_merge_heads(W1), b1, A1w, False),
        (_merge_heads(W2), b2, A2w, True),
    )
    xin = x
    for W, b, Aw, final in layers:
        h, q, M = _tc_dense(xin, W, b.reshape(1, F), _wq(Aw))
        T = _tc_table(h, q, M)
        oc = F if final else ACC
        xin = _sc_edge(T, dstp, soffp, est, final).reshape(NPAD, oc)
    return xin[:N]


# speedup vs baseline: 117.5601x; 1.3381x over previous
import functools

import jax
import jax.numpy as jnp
from jax import lax
from jax.experimental import pallas as pl
from jax.experimental.pallas import tpu as pltpu
from jax.experimental.pallas import tpu_sc as plsc

N = 10000
NH = 4
F = 64
L = 16
NC, NS = 2, 16
NW = NC * NS
NPW = 320
NPAD = NW * NPW
BB = 128
TROW = 128
ACC = 224
BN = 1000


def _tc_dense(xin, W, b2d, wq):
    Fin = xin.shape[1]
    grid = N // BN

    def body(x_ref, w_ref, b_ref, wq_ref, h_ref, q_ref, m_ref, macc):
        i = pl.program_id(0)
        h = jnp.dot(x_ref[...], w_ref[...].T, preferred_element_type=jnp.float32)
        h = h + b_ref[...]
        h = jnp.where(h >= 0.0, h, 0.2 * h)
        h_ref[...] = h
        q = jnp.dot(h, wq_ref[...].T, preferred_element_type=jnp.float32)
        q_ref[...] = q
        bm = jnp.max(q, axis=0, keepdims=True)

        @pl.when(i == 0)
        def _():
            macc[0:1, 0:8] = bm

        @pl.when(i > 0)
        def _():
            macc[0:1, 0:8] = jnp.maximum(macc[0:1, 0:8], bm)

        @pl.when(i == grid - 1)
        def _():
            m_ref[...] = macc[0:1, 0:8]

    return pl.pallas_call(
        body,
        grid=(grid,),
        in_specs=[
            pl.BlockSpec((BN, Fin), lambda i: (i, 0)),
            pl.BlockSpec((F, Fin), lambda i: (0, 0)),
            pl.BlockSpec((1, F), lambda i: (0, 0)),
            pl.BlockSpec((8, F), lambda i: (0, 0)),
        ],
        out_specs=[
            pl.BlockSpec((BN, F), lambda i: (i, 0)),
            pl.BlockSpec((BN, 8), lambda i: (i, 0)),
            pl.BlockSpec((1, 8), lambda i: (0, 0)),
        ],
        out_shape=[
            jax.ShapeDtypeStruct((N, F), jnp.float32),
            jax.ShapeDtypeStruct((N, 8), jnp.float32),
            jax.ShapeDtypeStruct((1, 8), jnp.float32),
        ],
        scratch_shapes=[pltpu.VMEM((8, 128), jnp.float32)],
    )(xin, W, b2d, wq)


def _tc_table(h, q, M):
    grid = N // BN

    def body(h_ref, q_ref, m_ref, t_ref):
        g = jnp.exp(q_ref[...] - m_ref[...])
        p0 = jnp.broadcast_to(g[:, 0:1], (BN, L))
        p1 = jnp.broadcast_to(g[:, 1:2], (BN, L))
        pad = jnp.zeros((BN, TROW - 2 * L - F), jnp.float32)
        t_ref[...] = jnp.concatenate([p0, p1, h_ref[...], pad], axis=1)

    return pl.pallas_call(
        body,
        grid=(grid,),
        in_specs=[
            pl.BlockSpec((BN, F), lambda i: (i, 0)),
            pl.BlockSpec((BN, 8), lambda i: (i, 0)),
            pl.BlockSpec((1, 8), lambda i: (0, 0)),
        ],
        out_specs=pl.BlockSpec((BN, TROW), lambda i: (i, 0)),
        out_shape=jax.ShapeDtypeStruct((N, TROW), jnp.float32),
    )(h, q, M)


def _sc_edge(table, dstp, soffp, est, final):
    OC = F if final else ACC
    mesh = plsc.VectorSubcoreMesh(core_axis_name="c", subcore_axis_name="s")

    @functools.partial(
        pl.kernel,
        out_type=jax.ShapeDtypeStruct((NPAD * OC,), jnp.float32),
        mesh=mesh,
        scratch_types=[
            pltpu.VMEM((2, BB), jnp.int32),
            pltpu.VMEM((2, BB, TROW), jnp.float32),
            pltpu.VMEM((2 * (BB + L),), jnp.int32),
            pltpu.VMEM((48,), jnp.int32),
            pltpu.VMEM((NPW * ACC,), jnp.float32),
            pltpu.VMEM((NPW * F if final else L,), jnp.float32),
            pltpu.SemaphoreType.DMA((2,)),
            pltpu.SemaphoreType.DMA,
        ],
    )
    def k(table_hbm, dst_hbm, soff_hbm, est_hbm, out_hbm,
          idx_v, stage_v, soff_v, est_v, outb, outc, sem, isem):
        wid = lax.axis_index("c") * NS + lax.axis_index("s")
        n0 = wid * NPW
        pltpu.sync_copy(est_hbm, est_v)
        e0 = est_v[pl.ds(wid, L)][0]
        e1 = est_v[pl.ds(wid + 1, L)][0]
        e0a = (e0 // 8) * 8
        joff = e0 - e0a
        cnt = e1 - e0a
        nb = lax.div(cnt + BB - 1, BB)
        zero = jnp.zeros((L,), jnp.float32)
        io = lax.iota(jnp.int32, L)

        def zrow(r, c):
            for qq in range(ACC // L):
                outb[pl.ds(r * ACC + qq * L, L)] = zero
            return c
        lax.fori_loop(0, NPW, zrow, 0)

        def idesc(b):
            slot = lax.rem(b, 2)
            eb = e0a + b * BB
            return (
                pltpu.make_async_copy(
                    dst_hbm.at[pl.ds(eb, BB)], idx_v.at[slot], isem),
                pltpu.make_async_copy(
                    soff_hbm.at[pl.ds(eb, BB)],
                    soff_v.at[pl.ds(slot * (BB + L), BB)], isem),
            )

        def gdesc(b):
            slot = lax.rem(b, 2)
            return pltpu.make_async_copy(
                table_hbm.at[idx_v.at[slot]], stage_v.at[slot], sem.at[slot])

        @pl.when(nb > 0)
        def _():
            d0, d1 = idesc(0)
            d0.start(); d1.start(); d0.wait(); d1.wait()
            gdesc(0).start()

        @pl.when(nb > 1)
        def _():
            d0, d1 = idesc(1)
            d0.start(); d1.start()

        def batch(b, c):
            slot = lax.rem(b, 2)

            @pl.when(b + 1 < nb)
            def _():
                d0, d1 = idesc(b + 1)
                d0.wait(); d1.wait()
                gdesc(b + 1).start()

            gdesc(b).wait()

            @pl.when(b + 2 < nb)
            def _():
                d0, d1 = idesc(b + 2)
                d0.start(); d1.start()

            js = jnp.where(b == 0, joff, 0)
            je = jnp.minimum(cnt - b * BB, BB)

            def body1(j, base):
                rg2 = stage_v[slot, j, pl.ds(0, L)]
                rg3 = stage_v[slot, j, pl.ds(L, L)]
                sv = jnp.where(io == 0, rg2,
                               jnp.where(io == 1, rg3,
                                         jnp.where(io == 2, 1.0, 0.0)))
                for qq in range(4):
                    rh = stage_v[slot, j, pl.ds(2 * L + qq * L, L)]
                    plsc.addupdate(outb.at[pl.ds(base + qq * L, L)], rh)
                    plsc.addupdate(
                        outb.at[pl.ds(base + F + qq * L, L)], rg2 * rh)
                    plsc.addupdate(
                        outb.at[pl.ds(base + 2 * F + qq * L, L)], rg3 * rh)
                plsc.addupdate(outb.at[pl.ds(base + 3 * F, L)], sv)

            U = 4
            nmain = lax.div(je - js, U)

            def edge4(k, c2):
                jb = js + k * U
                chunk = soff_v[pl.ds(slot * (BB + L) + jb, L)]
                for u in range(U):
                    body1(jb + u, chunk[u])
                return c2

            lax.fori_loop(0, nmain, edge4, c)

            def edge1(j, c2):
                base = soff_v[pl.ds(slot * (BB + L) + j, L)][0]
                body1(j, base)
                return c2

            lax.fori_loop(js + nmain * U, je, edge1, c)
            return c

        lax.fori_loop(0, nb, batch, 0)

        lane0 = jnp.zeros((L,), jnp.int32)
        lane1 = jnp.full((L,), 1, jnp.int32)
        lane2 = jnp.full((L,), 2, jnp.int32)

        def nrow(r, c):
            base = r * ACC
            sv = outb[pl.ds(base + 3 * F, L)]
            s2 = jnp.maximum(sv.at[lane0].get(mode="promise_in_bounds"),
                             1e-30)
            s3 = jnp.maximum(sv.at[lane1].get(mode="promise_in_bounds"),
                             1e-30)
            dg = jnp.maximum(sv.at[lane2].get(mode="promise_in_bounds"),
                             1.0)
            if final:
                for qq in range(4):
                    au = outb[pl.ds(base + qq * L, L)]
                    a2 = outb[pl.ds(base + F + qq * L, L)]
                    a3 = outb[pl.ds(base + 2 * F + qq * L, L)]
                    v = (2.0 * au / dg + a2 / s2 + a3 / s3) * 0.25
                    outc[pl.ds(r * F + qq * L, L)] = jnp.maximum(v, 0.0)
            else:
                for qq in range(4):
                    o0 = base + qq * L
                    outb[pl.ds(o0, L)] = jnp.maximum(
                        outb[pl.ds(o0, L)] / dg, 0.0)
                    outb[pl.ds(o0 + F, L)] = jnp.maximum(
                        outb[pl.ds(o0 + F, L)] / s2, 0.0)
                    outb[pl.ds(o0 + 2 * F, L)] = jnp.maximum(
                        outb[pl.ds(o0 + 2 * F, L)] / s3, 0.0)
            return c
        lax.fori_loop(0, NPW, nrow, 0)

        if final:
            pltpu.sync_copy(outc, out_hbm.at[pl.ds(n0 * F, NPW * F)])
        else:
            pltpu.sync_copy(outb, out_hbm.at[pl.ds(n0 * ACC, NPW * ACC)])

    return k(table, dstp, soffp, est)


def _merge_heads(W):
    Wr = W.reshape(-1, F, NH)
    Wm = jnp.concatenate(
        [Wr[:, :, 0] + Wr[:, :, 1], Wr[:, :, 2], Wr[:, :, 3]], axis=1)
    return jnp.pad(Wm, ((0, 0), (0, ACC - 3 * F)))


def _wq(Aw):
    w = Aw[2:4, :F] + Aw[2:4, F:]
    return jnp.pad(w, ((0, 6), (0, 0)))


def kernel(x, edge_index, W0, b0, W1, b1, W2, b2,
           A0w, A0b, A1w, A1b, A2w, A2b):
    src = edge_index[0].astype(jnp.int32)
    dst = edge_index[1].astype(jnp.int32)
    E = src.shape[0]
    bounds = jnp.arange(NW + 1, dtype=jnp.int32) * NPW
    est = jnp.searchsorted(src, bounds, side="left").astype(jnp.int32)
    est = jnp.pad(est, (0, 48 - (NW + 1)), constant_values=E)
    dstp = jnp.pad(dst, (0, BB))
    soffp = jnp.pad((src % NPW) * ACC, (0, BB))

    layers = (
        (W0, b0, A0w, False),
        (_merge_heads(W1), b1, A1w, False),
        (_merge_heads(W2), b2, A2w, True),
    )
    xin = x
    for W, b, Aw, final in layers:
        h, q, M = _tc_dense(xin, W, b.reshape(1, F), _wq(Aw))
        T = _tc_table(h, q, M)
        oc = F if final else ACC
        xin = _sc_edge(T, dstp, soffp, est, final).reshape(NPAD, oc)
    return xin[:N]
